# baseline (speedup 1.0000x reference)
import functools

import jax
import jax.numpy as jnp
from jax import lax
from jax.experimental import pallas as pl
from jax.experimental.pallas import tpu as pltpu
from jax.experimental.pallas import tpu_sc as plsc

_C = 2000
_H = 32
_E = 32
_B = 64
_SEQ = 6
_NSTEP = _SEQ - 1
_NIDX = _NSTEP * _B
_PAD = 384
_CW = 2048
_EW = 128


def _sc_gather_body(gtab, etab, idxg, idxe, gr_out, em_out,
                    idxg_v, idxe_v, rows_v, erows_v, sem):
    nc = plsc.get_sparse_core_info().num_cores
    wid = lax.axis_index("s") * nc + lax.axis_index("c")
    base = wid * 24
    pltpu.sync_copy(idxg.at[pl.ds(base, 24)], idxg_v)
    pltpu.async_copy(gtab.at[idxg_v], rows_v, sem).wait()
    pltpu.sync_copy(rows_v, gr_out.at[pl.ds(base, 24)])
    pltpu.sync_copy(idxe.at[pl.ds(base, 24)], idxe_v)
    pltpu.async_copy(etab.at[idxe_v], erows_v, sem).wait()
    pltpu.sync_copy(erows_v, em_out.at[pl.ds(base, 24)])


def _sc_gather(gtab, etab, idxg, idxe):
    mesh = plsc.VectorSubcoreMesh(core_axis_name="c", subcore_axis_name="s")
    fn = functools.partial(
        pl.kernel,
        mesh=mesh,
        out_type=[
            jax.ShapeDtypeStruct((2 * _PAD, _CW), jnp.float32),
            jax.ShapeDtypeStruct((2 * _PAD, _EW), jnp.float32),
        ],
        scratch_types=[
            pltpu.VMEM((24,), jnp.int32),
            pltpu.VMEM((24,), jnp.int32),
            pltpu.VMEM((24, _CW), jnp.float32),
            pltpu.VMEM((24, _EW), jnp.float32),
            pltpu.SemaphoreType.DMA,
        ],
    )(_sc_gather_body)
    return fn(gtab, etab, idxg, idxe)


def _tc_body(qs, reT, ecqT, adj_all, rev_all, embcT, eaw,
             WA, b01, WU, WV, W2T, b2c, g01, bt01,
             WEAo, bEA, WG, bg,
             fsW1, fsb1, fsW2, fsb2, fsg, fsbt,
             wp, bpv, out_ref, ht_s, u_cache):
    f32 = jnp.float32
    dot = functools.partial(jnp.dot, preferred_element_type=f32)

    def relu(x):
        return jnp.maximum(x, 0.0)

    ht_s[...] = jnp.zeros((_B * _H, _CW), f32)
    WAv, WUv, WVv, W2Tv = WA[...], WU[...], WV[...], W2T[...]
    WEAv, WGv = WEAo[...], WG[...]
    b01v, b2cv, bEAv, bgv = b01[...], b2c[...], bEA[...], bg[...]
    g01v, bt01v = g01[...], bt01[...]
    fsW1v, fsW2v = fsW1[...], fsW2[...]
    fsb1v, fsb2v, fsgv, fsbtv = fsb1[...], fsb2[...], fsg[...], fsbt[...]
    wpv, eawv = wp[...], eaw[...]
    bf16 = jnp.bfloat16
    WUb, W2Tb = WUv.astype(bf16), W2Tv.astype(bf16)
    WEAb, WGb = WEAv.astype(bf16), WGv.astype(bf16)
    ecv = dot(WVv, embcT[...])
    lane = lax.broadcasted_iota(jnp.int32, (1, _CW), 1)
    valid = (lane < _C).astype(f32)
    bcol = lax.broadcasted_iota(jnp.int32, (_B, 1), 0)
    brow = lax.broadcasted_iota(jnp.int32, (1, _B), 1)
    inv_n = f32(1.0 / (_B * _C))
    WA_h = WAv[:, :_H]

    def _next_consts(j):
        reTj = reT[:, j * _B:(j + 1) * _B]
        ecqTj = ecqT[:, j * _B:(j + 1) * _B]
        sext = dot(WAv[:, _H:], reTj) + b01v
        d01T = dot(WVv, reTj - ecqTj)
        return sext, d01T

    zs = jnp.zeros((2 * _H, 1), f32)

    sext0, d01T0 = _next_consts(0)

    def _pa0(b, carry):
        ssum, ssq = carry
        ohc = (bcol == b).astype(f32)
        s_c = dot(sext0, ohc)
        d_c = dot(d01T0, ohc)
        mq = (lane == qs[b, 0]).astype(f32)
        zT = s_c + ecv + d_c * mq
        uT = relu(dot(W2Tb, relu(zT).astype(bf16)) + b2cv) * valid
        u_cache[pl.ds(b * 2 * _H, 2 * _H), :] = uT
        return (ssum + jnp.sum(uT, axis=1, keepdims=True),
                ssq + jnp.sum(uT * uT, axis=1, keepdims=True))

    ssum, ssq = lax.fori_loop(0, _B, _pa0, (zs, zs))

    selfhT = jnp.zeros((_H, _B), f32)
    for i in range(_NSTEP):
        mu01 = ssum * inv_n
        var01 = ssq * inv_n - mu01 * mu01
        sc01 = g01v * lax.rsqrt(var01 + 1e-5)
        sh01 = bt01v - mu01 * sc01

        reTi = reT[:, i * _B:(i + 1) * _B]
        shT = jnp.concatenate([selfhT, reTi], axis=0)
        h1 = relu(dot(fsW1v, shT) + fsb1v)
        h2 = relu(dot(fsW2v, h1) + fsb2v)
        mu = jnp.mean(h2, axis=1, keepdims=True)
        va = jnp.mean((h2 - mu) * (h2 - mu), axis=1, keepdims=True)
        sfT = (h2 - mu) * lax.rsqrt(va + 1e-5) * fsgv + fsbtv

        last = i == _NSTEP - 1
        if not last:
            sextn, d01Tn = _next_consts(i + 1)

        def _merged(b, carry):
            pacc, snext, ssum2, ssq2 = carry
            X = ht_s[pl.ds(b * _H, _H), :]
            uT = u_cache[pl.ds(b * 2 * _H, 2 * _H), :]
            ohc = (bcol == b).astype(f32)
            mq = (lane == qs[b, i]).astype(f32)
            n01 = uT * sc01 + sh01
            n0 = n01[:_H, :]
            n1 = n01[_H:, :]
            r_adj = adj_all[pl.ds(i * _B + b, 1), :]
            r_rev = rev_all[pl.ds(i * _B + b, 1), :]
            neigh = r_adj * n0 + r_rev * n1
            sf_c = dot(sfT, ohc)
            mn = neigh + mq * (sf_c - neigh)
            eT = dot(WEAb, mn.astype(bf16)) + bEAv
            eg = jax.nn.sigmoid(eT[:_H, :])
            ad = jnp.tanh(eT[_H:, :])
            m2 = mn - eawv * eg * mn + eawv * ad
            XT = jnp.concatenate([m2, X], axis=0).astype(bf16)
            G = dot(WGb, XT) + bgv
            r = jax.nn.sigmoid(G[:_H, :])
            zg = jax.nn.sigmoid(G[_H:2 * _H, :])
            n = jnp.tanh(G[2 * _H:3 * _H, :] + r * G[3 * _H:, :])
            hn = (1.0 - zg) * n + zg * X
            ht_s[pl.ds(b * _H, _H), :] = hn
            mqn = (lane == qs[b, i + 1]).astype(f32)
            col = jnp.sum(hn * mqn, axis=1, keepdims=True)
            ohr = (brow == b).astype(f32)
            snext = snext + col * ohr
            pv = jnp.sum(col * wpv, axis=0, keepdims=True)
            pacc = pacc + pv * ohc
            if not last:
                s_c2 = dot(WA_h, col) + dot(sextn, ohc)
                d_c2 = dot(d01Tn, ohc)
                zT2 = dot(WUb, hn.astype(bf16)) + s_c2 + ecv + d_c2 * mqn
                uT2 = relu(dot(W2Tb, relu(zT2).astype(bf16)) + b2cv) * valid
                u_cache[pl.ds(b * 2 * _H, 2 * _H), :] = uT2
                ssum2 = ssum2 + jnp.sum(uT2, axis=1, keepdims=True)
                ssq2 = ssq2 + jnp.sum(uT2 * uT2, axis=1, keepdims=True)
            return pacc, snext, ssum2, ssq2

        pacc, selfhT, ssum, ssq = lax.fori_loop(
            0, _B, _merged,
            (jnp.zeros((_B, 1), f32), jnp.zeros((_H, _B), f32), zs, zs))
        out_ref[:, i:i + 1] = jax.nn.sigmoid(pacc + bpv[...])


def _tc_call(args):
    return pl.pallas_call(
        _tc_body,
        out_shape=jax.ShapeDtypeStruct((_B, _NSTEP), jnp.float32),
        in_specs=[pl.BlockSpec(memory_space=pltpu.SMEM)]
        + [pl.BlockSpec(memory_space=pltpu.VMEM)] * 26,
        out_specs=pl.BlockSpec(memory_space=pltpu.VMEM),
        scratch_shapes=[
            pltpu.VMEM((_B * _H, _CW), jnp.float32),
            pltpu.VMEM((_B * 2 * _H, _CW), jnp.float32),
        ],
    )(*args)


def kernel(features, questions, emb_x, emb_c, graph, fs_W1, fs_b1, fs_W2,
           fs_b2, fs_g, fs_bt, fn0_W1, fn0_b1, fn0_W2, fn0_b2, fn0_g,
           fn0_bt, fn1_W1, fn1_b1, fn1_W2, fn1_b2, fn1_g, fn1_bt, ea_w,
           ea_We, ea_be, ea_Wa, ea_ba, gru_Wih, gru_Whh, gru_bih, gru_bhh,
           Wp, bp):
    f32 = jnp.float32
    qi = questions.astype(jnp.int32)
    xi = features.astype(jnp.int32)

    qt_flat = qi[:, :_NSTEP].T.reshape(_NIDX)
    xt_flat = xi[:, :_NSTEP].T.reshape(_NIDX)
    padz = jnp.zeros((_PAD - _NIDX,), jnp.int32)
    idxg = jnp.concatenate([qt_flat, padz, qt_flat + _C, padz])
    idxe = jnp.concatenate([xt_flat, padz, qt_flat + 2 * _C, padz])

    gtab = jnp.pad(jnp.concatenate([graph, graph.T], axis=0),
                   ((0, 0), (0, _CW - _C)))
    etab = jnp.pad(jnp.concatenate([emb_x, emb_c[:_C, :]], axis=0),
                   ((0, 0), (0, _EW - _E)))

    gr_out, em_out = _sc_gather(gtab, etab, idxg, idxe)
    adj_all = gr_out[:_NIDX]
    rev_all = gr_out[_PAD:_PAD + _NIDX]
    reT = em_out[:_NIDX, :_E].T
    ecqT = em_out[_PAD:_PAD + _NIDX, :_E].T

    WA = jnp.concatenate([fn0_W1[:, :2 * _H], fn1_W1[:, :2 * _H]], axis=0)
    WU = jnp.concatenate([fn0_W1[:, 2 * _H:3 * _H],
                          fn1_W1[:, 2 * _H:3 * _H]], axis=0)
    WV = jnp.concatenate([fn0_W1[:, 3 * _H:], fn1_W1[:, 3 * _H:]], axis=0)
    b01 = jnp.concatenate([fn0_b1, fn1_b1]).reshape(2 * _H, 1)
    zH = jnp.zeros((_H, _H), f32)
    W2T = jnp.concatenate([
        jnp.concatenate([fn0_W2, zH], axis=1),
        jnp.concatenate([zH, fn1_W2], axis=1)], axis=0)
    b2c = jnp.concatenate([fn0_b2, fn1_b2]).reshape(2 * _H, 1)
    g01 = jnp.concatenate([fn0_g, fn1_g]).reshape(2 * _H, 1)
    bt01 = jnp.concatenate([fn0_bt, fn1_bt]).reshape(2 * _H, 1)
    WEAo = jnp.concatenate([ea_We, ea_Wa], axis=0)
    bEA = jnp.concatenate([ea_be, ea_ba]).reshape(2 * _H, 1)
    zH2 = jnp.zeros((_H, _H), f32)
    WG = jnp.concatenate([
        jnp.concatenate([gru_Wih[:_H], gru_Whh[:_H]], axis=1),
        jnp.concatenate([gru_Wih[_H:2 * _H], gru_Whh[_H:2 * _H]], axis=1),
        jnp.concatenate([gru_Wih[2 * _H:], zH2], axis=1),
        jnp.concatenate([zH2, gru_Whh[2 * _H:]], axis=1),
    ], axis=0)
    bg = jnp.concatenate([
        gru_bih[:2 * _H] + gru_bhh[:2 * _H],
        gru_bih[2 * _H:], gru_bhh[2 * _H:]]).reshape(4 * _H, 1)

    embcT = jnp.pad(emb_c[:_C, :].T, ((0, 0), (0, _CW - _C)))
    eaw_r = jnp.pad(ea_w, (0, _CW - _C)).reshape(1, _CW)

    args = (qi, reT, ecqT, adj_all, rev_all, embcT, eaw_r,
            WA, b01, WU, WV, W2T, b2c, g01, bt01,
            WEAo, bEA, WG, bg,
            fs_W1, fs_b1.reshape(_H, 1), fs_W2, fs_b2.reshape(_H, 1),
            fs_g.reshape(_H, 1), fs_bt.reshape(_H, 1),
            Wp.reshape(_H, 1), bp.reshape(1, 1))
    return _tc_call(args)


# speedup vs baseline: 1.0238x; 1.0238x over previous
import functools

import jax
import jax.numpy as jnp
from jax import lax
from jax.experimental import pallas as pl
from jax.experimental.pallas import tpu as pltpu
from jax.experimental.pallas import tpu_sc as plsc

_C = 2000
_H = 32
_E = 32
_B = 64
_SEQ = 6
_NSTEP = _SEQ - 1
_NIDX = _NSTEP * _B
_PAD = 384
_CW = 2048
_EW = 128


def _sc_gather_body(gtab, etab, idxg, idxe, gr_out, em_out,
                    idxg_v, idxe_v, rows_v, erows_v, sem):
    nc = plsc.get_sparse_core_info().num_cores
    wid = lax.axis_index("s") * nc + lax.axis_index("c")
    base = wid * 24
    pltpu.sync_copy(idxg.at[pl.ds(base, 24)], idxg_v)
    pltpu.async_copy(gtab.at[idxg_v], rows_v, sem).wait()
    pltpu.sync_copy(rows_v, gr_out.at[pl.ds(base, 24)])
    pltpu.sync_copy(idxe.at[pl.ds(base, 24)], idxe_v)
    pltpu.async_copy(etab.at[idxe_v], erows_v, sem).wait()
    pltpu.sync_copy(erows_v, em_out.at[pl.ds(base, 24)])


def _sc_gather(gtab, etab, idxg, idxe):
    mesh = plsc.VectorSubcoreMesh(core_axis_name="c", subcore_axis_name="s")
    fn = functools.partial(
        pl.kernel,
        mesh=mesh,
        out_type=[
            jax.ShapeDtypeStruct((2 * _PAD, _CW), jnp.float32),
            jax.ShapeDtypeStruct((2 * _PAD, _EW), jnp.float32),
        ],
        scratch_types=[
            pltpu.VMEM((24,), jnp.int32),
            pltpu.VMEM((24,), jnp.int32),
            pltpu.VMEM((24, _CW), jnp.float32),
            pltpu.VMEM((24, _EW), jnp.float32),
            pltpu.SemaphoreType.DMA,
        ],
    )(_sc_gather_body)
    return fn(gtab, etab, idxg, idxe)


def _tc_body(qs, reT, ecqT, gr_all, embcT, eaw,
             WA, b01, WU, WV, W2T, b2c, g01, bt01,
             WEAo, bEA, WG, bg,
             fsW1, fsb1, fsW2, fsb2, fsg, fsbt,
             wp, bpv, out_ref, ht_s, u_cache):
    f32 = jnp.float32
    dot = functools.partial(jnp.dot, preferred_element_type=f32)

    def relu(x):
        return jnp.maximum(x, 0.0)

    ht_s[...] = jnp.zeros((_B * _H, _CW), f32)
    WAv, WUv, WVv, W2Tv = WA[...], WU[...], WV[...], W2T[...]
    WEAv, WGv = WEAo[...], WG[...]
    b01v, b2cv, bEAv, bgv = b01[...], b2c[...], bEA[...], bg[...]
    g01v, bt01v = g01[...], bt01[...]
    fsW1v, fsW2v = fsW1[...], fsW2[...]
    fsb1v, fsb2v, fsgv, fsbtv = fsb1[...], fsb2[...], fsg[...], fsbt[...]
    wpv, eawv = wp[...], eaw[...]
    ecv = dot(WVv, embcT[...])
    lane = lax.broadcasted_iota(jnp.int32, (1, _CW), 1)
    valid = (lane < _C).astype(f32)
    bcol = lax.broadcasted_iota(jnp.int32, (_B, 1), 0)
    brow = lax.broadcasted_iota(jnp.int32, (1, _B), 1)
    inv_n = f32(1.0 / (_B * _C))
    WA_h = WAv[:, :_H]

    def _next_consts(j):
        reTj = reT[:, j * _B:(j + 1) * _B]
        ecqTj = ecqT[:, j * _B:(j + 1) * _B]
        sext = dot(WAv[:, _H:], reTj) + b01v
        d01T = dot(WVv, reTj - ecqTj)
        return sext, d01T

    zs = jnp.zeros((2 * _H, 1), f32)

    sext0, d01T0 = _next_consts(0)

    def _pa0(b, carry):
        ssum, ssq = carry
        ohc = (bcol == b).astype(f32)
        s_c = dot(sext0, ohc)
        d_c = dot(d01T0, ohc)
        mq = (lane == qs[b, 0]).astype(f32)
        zT = s_c + ecv + d_c * mq
        uT = relu(dot(W2Tv, relu(zT)) + b2cv) * valid
        u_cache[pl.ds(b * 2 * _H, 2 * _H), :] = uT
        return (ssum + jnp.sum(uT, axis=1, keepdims=True),
                ssq + jnp.sum(uT * uT, axis=1, keepdims=True))

    ssum, ssq = lax.fori_loop(0, _B, _pa0, (zs, zs))

    selfhT = jnp.zeros((_H, _B), f32)
    for i in range(_NSTEP):
        mu01 = ssum * inv_n
        var01 = ssq * inv_n - mu01 * mu01
        sc01 = g01v * lax.rsqrt(var01 + 1e-5)
        sh01 = bt01v - mu01 * sc01

        reTi = reT[:, i * _B:(i + 1) * _B]
        shT = jnp.concatenate([selfhT, reTi], axis=0)
        h1 = relu(dot(fsW1v, shT) + fsb1v)
        h2 = relu(dot(fsW2v, h1) + fsb2v)
        mu = jnp.mean(h2, axis=1, keepdims=True)
        va = jnp.mean((h2 - mu) * (h2 - mu), axis=1, keepdims=True)
        sfT = (h2 - mu) * lax.rsqrt(va + 1e-5) * fsgv + fsbtv

        last = i == _NSTEP - 1
        if not last:
            sextn, d01Tn = _next_consts(i + 1)

        def _merged(b, carry):
            pacc, snext, ssum2, ssq2 = carry
            X = ht_s[pl.ds(b * _H, _H), :]
            uT = u_cache[pl.ds(b * 2 * _H, 2 * _H), :]
            ohc = (bcol == b).astype(f32)
            mq = (lane == qs[b, i]).astype(f32)
            n01 = uT * sc01 + sh01
            n0 = n01[:_H, :]
            n1 = n01[_H:, :]
            r_adj = gr_all[pl.ds(i * _B + b, 1), :]
            r_rev = gr_all[pl.ds(_PAD + i * _B + b, 1), :]
            neigh = r_adj * n0 + r_rev * n1
            sf_c = dot(sfT, ohc)
            mn = neigh + mq * (sf_c - neigh)
            eT = dot(WEAv, mn) + bEAv
            eg = jax.nn.sigmoid(eT[:_H, :])
            ad = jnp.tanh(eT[_H:, :])
            m2 = mn - eawv * eg * mn + eawv * ad
            XT = jnp.concatenate([m2, X], axis=0)
            G = dot(WGv, XT) + bgv
            r = jax.nn.sigmoid(G[:_H, :])
            zg = jax.nn.sigmoid(G[_H:2 * _H, :])
            n = jnp.tanh(G[2 * _H:3 * _H, :] + r * G[3 * _H:, :])
            hn = (1.0 - zg) * n + zg * X
            ht_s[pl.ds(b * _H, _H), :] = hn
            mqn = (lane == qs[b, i + 1]).astype(f32)
            col = jnp.sum(hn * mqn, axis=1, keepdims=True)
            ohr = (brow == b).astype(f32)
            snext = snext + col * ohr
            pv = jnp.sum(col * wpv, axis=0, keepdims=True)
            pacc = pacc + pv * ohc
            if not last:
                s_c2 = dot(WA_h, col) + dot(sextn, ohc)
                d_c2 = dot(d01Tn, ohc)
                zT2 = dot(WUv, hn) + s_c2 + ecv + d_c2 * mqn
                uT2 = relu(dot(W2Tv, relu(zT2)) + b2cv) * valid
                u_cache[pl.ds(b * 2 * _H, 2 * _H), :] = uT2
                ssum2 = ssum2 + jnp.sum(uT2, axis=1, keepdims=True)
                ssq2 = ssq2 + jnp.sum(uT2 * uT2, axis=1, keepdims=True)
            return pacc, snext, ssum2, ssq2

        pacc, selfhT, ssum, ssq = lax.fori_loop(
            0, _B, _merged,
            (jnp.zeros((_B, 1), f32), jnp.zeros((_H, _B), f32), zs, zs))
        out_ref[:, i:i + 1] = jax.nn.sigmoid(pacc + bpv[...])


def _tc_call(args):
    return pl.pallas_call(
        _tc_body,
        out_shape=jax.ShapeDtypeStruct((_B, _NSTEP), jnp.float32),
        in_specs=[pl.BlockSpec(memory_space=pltpu.SMEM)]
        + [pl.BlockSpec(memory_space=pltpu.VMEM)] * 25,
        out_specs=pl.BlockSpec(memory_space=pltpu.VMEM),
        scratch_shapes=[
            pltpu.VMEM((_B * _H, _CW), jnp.float32),
            pltpu.VMEM((_B * 2 * _H, _CW), jnp.float32),
        ],
    )(*args)


def kernel(features, questions, emb_x, emb_c, graph, fs_W1, fs_b1, fs_W2,
           fs_b2, fs_g, fs_bt, fn0_W1, fn0_b1, fn0_W2, fn0_b2, fn0_g,
           fn0_bt, fn1_W1, fn1_b1, fn1_W2, fn1_b2, fn1_g, fn1_bt, ea_w,
           ea_We, ea_be, ea_Wa, ea_ba, gru_Wih, gru_Whh, gru_bih, gru_bhh,
           Wp, bp):
    f32 = jnp.float32
    qi = questions.astype(jnp.int32)
    xi = features.astype(jnp.int32)

    qt_flat = qi[:, :_NSTEP].T.reshape(_NIDX)
    xt_flat = xi[:, :_NSTEP].T.reshape(_NIDX)
    padz = jnp.zeros((_PAD - _NIDX,), jnp.int32)
    idxg = jnp.concatenate([qt_flat, padz, qt_flat + _C, padz])
    idxe = jnp.concatenate([xt_flat, padz, qt_flat + 2 * _C, padz])

    gtab = jnp.pad(jnp.concatenate([graph, graph.T], axis=0),
                   ((0, 0), (0, _CW - _C)))
    etab = jnp.pad(jnp.concatenate([emb_x, emb_c[:_C, :]], axis=0),
                   ((0, 0), (0, _EW - _E)))

    gr_out, em_out = _sc_gather(gtab, etab, idxg, idxe)
    reT = em_out[:_NIDX, :_E].T
    ecqT = em_out[_PAD:_PAD + _NIDX, :_E].T

    WA = jnp.concatenate([fn0_W1[:, :2 * _H], fn1_W1[:, :2 * _H]], axis=0)
    WU = jnp.concatenate([fn0_W1[:, 2 * _H:3 * _H],
                          fn1_W1[:, 2 * _H:3 * _H]], axis=0)
    WV = jnp.concatenate([fn0_W1[:, 3 * _H:], fn1_W1[:, 3 * _H:]], axis=0)
    b01 = jnp.concatenate([fn0_b1, fn1_b1]).reshape(2 * _H, 1)
    zH = jnp.zeros((_H, _H), f32)
    W2T = jnp.concatenate([
        jnp.concatenate([fn0_W2, zH], axis=1),
        jnp.concatenate([zH, fn1_W2], axis=1)], axis=0)
    b2c = jnp.concatenate([fn0_b2, fn1_b2]).reshape(2 * _H, 1)
    g01 = jnp.concatenate([fn0_g, fn1_g]).reshape(2 * _H, 1)
    bt01 = jnp.concatenate([fn0_bt, fn1_bt]).reshape(2 * _H, 1)
    WEAo = jnp.concatenate([ea_We, ea_Wa], axis=0)
    bEA = jnp.concatenate([ea_be, ea_ba]).reshape(2 * _H, 1)
    zH2 = jnp.zeros((_H, _H), f32)
    WG = jnp.concatenate([
        jnp.concatenate([gru_Wih[:_H], gru_Whh[:_H]], axis=1),
        jnp.concatenate([gru_Wih[_H:2 * _H], gru_Whh[_H:2 * _H]], axis=1),
        jnp.concatenate([gru_Wih[2 * _H:], zH2], axis=1),
        jnp.concatenate([zH2, gru_Whh[2 * _H:]], axis=1),
    ], axis=0)
    bg = jnp.concatenate([
        gru_bih[:2 * _H] + gru_bhh[:2 * _H],
        gru_bih[2 * _H:], gru_bhh[2 * _H:]]).reshape(4 * _H, 1)

    embcT = jnp.pad(emb_c[:_C, :].T, ((0, 0), (0, _CW - _C)))
    eaw_r = jnp.pad(ea_w, (0, _CW - _C)).reshape(1, _CW)

    args = (qi, reT, ecqT, gr_out, embcT, eaw_r,
            WA, b01, WU, WV, W2T, b2c, g01, bt01,
            WEAo, bEA, WG, bg,
            fs_W1, fs_b1.reshape(_H, 1), fs_W2, fs_b2.reshape(_H, 1),
            fs_g.reshape(_H, 1), fs_bt.reshape(_H, 1),
            Wp.reshape(_H, 1), bp.reshape(1, 1))
    return _tc_call(args)


# speedup vs baseline: 1.0414x; 1.0171x over previous
import functools

import jax
import jax.numpy as jnp
from jax import lax
from jax.experimental import pallas as pl
from jax.experimental.pallas import tpu as pltpu
from jax.experimental.pallas import tpu_sc as plsc

_C = 2000
_H = 32
_E = 32
_B = 64
_SEQ = 6
_NSTEP = _SEQ - 1
_NIDX = _NSTEP * _B
_PAD = 384
_PADR = 512
_CW = 2048
_EW = 128


def _sc_gather_body(gtabT, etab, idxr, idxe, rv_out, em_out,
                    idxr_v, idxe_v, rows_v, erows_v, sem):
    nc = plsc.get_sparse_core_info().num_cores
    wid = lax.axis_index("s") * nc + lax.axis_index("c")
    baser = wid * 16
    pltpu.sync_copy(idxr.at[pl.ds(baser, 16)], idxr_v)
    pltpu.async_copy(gtabT.at[idxr_v], rows_v, sem).wait()
    pltpu.sync_copy(rows_v, rv_out.at[pl.ds(baser, 16)])
    base = wid * 24
    pltpu.sync_copy(idxe.at[pl.ds(base, 24)], idxe_v)
    pltpu.async_copy(etab.at[idxe_v], erows_v, sem).wait()
    pltpu.sync_copy(erows_v, em_out.at[pl.ds(base, 24)])


def _sc_gather(gtabT, etab, idxr, idxe):
    mesh = plsc.VectorSubcoreMesh(core_axis_name="c", subcore_axis_name="s")
    fn = functools.partial(
        pl.kernel,
        mesh=mesh,
        out_type=[
            jax.ShapeDtypeStruct((_PADR, _CW), jnp.float32),
            jax.ShapeDtypeStruct((2 * _PAD, _EW), jnp.float32),
        ],
        scratch_types=[
            pltpu.VMEM((16,), jnp.int32),
            pltpu.VMEM((24,), jnp.int32),
            pltpu.VMEM((16, _CW), jnp.float32),
            pltpu.VMEM((24, _EW), jnp.float32),
            pltpu.SemaphoreType.DMA,
        ],
    )(_sc_gather_body)
    return fn(gtabT, etab, idxr, idxe)


def _tc_body(qs, reT, ecqT, rv_all, graph, embcT, eaw,
             WA, b01, WU, WV, W2T, b2c, g01, bt01,
             WEAo, bEA, WG, bg,
             fsW1, fsb1, fsW2, fsb2, fsg, fsbt,
             wp, bpv, out_ref, ht_s, u_cache):
    f32 = jnp.float32
    dot = functools.partial(jnp.dot, preferred_element_type=f32)

    def relu(x):
        return jnp.maximum(x, 0.0)

    ht_s[...] = jnp.zeros((_B * _H, _CW), f32)
    WAv, WUv, WVv, W2Tv = WA[...], WU[...], WV[...], W2T[...]
    WEAv, WGv = WEAo[...], WG[...]
    b01v, b2cv, bEAv, bgv = b01[...], b2c[...], bEA[...], bg[...]
    g01v, bt01v = g01[...], bt01[...]
    fsW1v, fsW2v = fsW1[...], fsW2[...]
    fsb1v, fsb2v, fsgv, fsbtv = fsb1[...], fsb2[...], fsg[...], fsbt[...]
    wpv, eawv = wp[...], eaw[...]
    ecv = dot(WVv, embcT[...])
    lane = lax.broadcasted_iota(jnp.int32, (1, _CW), 1)
    valid = (lane < _C).astype(f32)
    bcol = lax.broadcasted_iota(jnp.int32, (_B, 1), 0)
    brow = lax.broadcasted_iota(jnp.int32, (1, _B), 1)
    inv_n = f32(1.0 / (_B * _C))
    WA_h = WAv[:, :_H]
    zpad = jnp.zeros((1, _CW - _C), f32)

    def _next_consts(j):
        reTj = reT[:, j * _B:(j + 1) * _B]
        ecqTj = ecqT[:, j * _B:(j + 1) * _B]
        sext = dot(WAv[:, _H:], reTj) + b01v
        d01T = dot(WVv, reTj - ecqTj)
        return sext, d01T

    zs = jnp.zeros((2 * _H, 1), f32)

    sext0, d01T0 = _next_consts(0)

    def _pa0(b, carry):
        ssum, ssq = carry
        ohc = (bcol == b).astype(f32)
        s_c = dot(sext0, ohc)
        d_c = dot(d01T0, ohc)
        mq = (lane == qs[b, 0]).astype(f32)
        zT = s_c + ecv + d_c * mq
        uT = relu(dot(W2Tv, relu(zT)) + b2cv) * valid
        u_cache[pl.ds(b * 2 * _H, 2 * _H), :] = uT.astype(jnp.bfloat16)
        return (ssum + jnp.sum(uT, axis=1, keepdims=True),
                ssq + jnp.sum(uT * uT, axis=1, keepdims=True))

    ssum, ssq = lax.fori_loop(0, _B, _pa0, (zs, zs))

    selfhT = jnp.zeros((_H, _B), f32)
    for i in range(_NSTEP):
        mu01 = ssum * inv_n
        var01 = ssq * inv_n - mu01 * mu01
        sc01 = g01v * lax.rsqrt(var01 + 1e-5)
        sh01 = bt01v - mu01 * sc01

        reTi = reT[:, i * _B:(i + 1) * _B]
        shT = jnp.concatenate([selfhT, reTi], axis=0)
        h1 = relu(dot(fsW1v, shT) + fsb1v)
        h2 = relu(dot(fsW2v, h1) + fsb2v)
        mu = jnp.mean(h2, axis=1, keepdims=True)
        va = jnp.mean((h2 - mu) * (h2 - mu), axis=1, keepdims=True)
        sfT = (h2 - mu) * lax.rsqrt(va + 1e-5) * fsgv + fsbtv

        last = i == _NSTEP - 1
        if not last:
            sextn, d01Tn = _next_consts(i + 1)

        def _merged(b, carry):
            pacc, snext, ssum2, ssq2 = carry
            X = ht_s[pl.ds(b * _H, _H), :]
            uT = u_cache[pl.ds(b * 2 * _H, 2 * _H), :].astype(f32)
            ohc = (bcol == b).astype(f32)
            mq = (lane == qs[b, i]).astype(f32)
            n01 = uT * sc01 + sh01
            n0 = n01[:_H, :]
            n1 = n01[_H:, :]
            r_adj = jnp.concatenate(
                [graph[pl.ds(qs[b, i], 1), :], zpad], axis=1)
            r_rev = rv_all[pl.ds(i * _B + b, 1), :]
            neigh = r_adj * n0 + r_rev * n1
            sf_c = dot(sfT, ohc)
            mn = neigh + mq * (sf_c - neigh)
            eT = dot(WEAv, mn) + bEAv
            eg = jax.nn.sigmoid(eT[:_H, :])
            ad = jnp.tanh(eT[_H:, :])
            m2 = mn - eawv * eg * mn + eawv * ad
            XT = jnp.concatenate([m2, X], axis=0)
            G = dot(WGv, XT) + bgv
            r = jax.nn.sigmoid(G[:_H, :])
            zg = jax.nn.sigmoid(G[_H:2 * _H, :])
            n = jnp.tanh(G[2 * _H:3 * _H, :] + r * G[3 * _H:, :])
            hn = (1.0 - zg) * n + zg * X
            ht_s[pl.ds(b * _H, _H), :] = hn
            mqn = (lane == qs[b, i + 1]).astype(f32)
            col = jnp.sum(hn * mqn, axis=1, keepdims=True)
            ohr = (brow == b).astype(f32)
            snext = snext + col * ohr
            pv = jnp.sum(col * wpv, axis=0, keepdims=True)
            pacc = pacc + pv * ohc
            if not last:
                s_c2 = dot(WA_h, col) + dot(sextn, ohc)
                d_c2 = dot(d01Tn, ohc)
                zT2 = dot(WUv, hn) + s_c2 + ecv + d_c2 * mqn
                uT2 = relu(dot(W2Tv, relu(zT2)) + b2cv) * valid
                u_cache[pl.ds(b * 2 * _H, 2 * _H), :] = uT2.astype(jnp.bfloat16)
                ssum2 = ssum2 + jnp.sum(uT2, axis=1, keepdims=True)
                ssq2 = ssq2 + jnp.sum(uT2 * uT2, axis=1, keepdims=True)
            return pacc, snext, ssum2, ssq2

        pacc, selfhT, ssum, ssq = lax.fori_loop(
            0, _B, _merged,
            (jnp.zeros((_B, 1), f32), jnp.zeros((_H, _B), f32), zs, zs))
        out_ref[:, i:i + 1] = jax.nn.sigmoid(pacc + bpv[...])


def _tc_call(args):
    return pl.pallas_call(
        _tc_body,
        out_shape=jax.ShapeDtypeStruct((_B, _NSTEP), jnp.float32),
        in_specs=[pl.BlockSpec(memory_space=pltpu.SMEM)]
        + [pl.BlockSpec(memory_space=pltpu.VMEM)] * 26,
        out_specs=pl.BlockSpec(memory_space=pltpu.VMEM),
        compiler_params=pltpu.CompilerParams(
            vmem_limit_bytes=100 * 1024 * 1024),
        scratch_shapes=[
            pltpu.VMEM((_B * _H, _CW), jnp.float32),
            pltpu.VMEM((_B * 2 * _H, _CW), jnp.bfloat16),
        ],
    )(*args)


def kernel(features, questions, emb_x, emb_c, graph, fs_W1, fs_b1, fs_W2,
           fs_b2, fs_g, fs_bt, fn0_W1, fn0_b1, fn0_W2, fn0_b2, fn0_g,
           fn0_bt, fn1_W1, fn1_b1, fn1_W2, fn1_b2, fn1_g, fn1_bt, ea_w,
           ea_We, ea_be, ea_Wa, ea_ba, gru_Wih, gru_Whh, gru_bih, gru_bhh,
           Wp, bp):
    f32 = jnp.float32
    qi = questions.astype(jnp.int32)
    xi = features.astype(jnp.int32)

    qt_flat = qi[:, :_NSTEP].T.reshape(_NIDX)
    xt_flat = xi[:, :_NSTEP].T.reshape(_NIDX)
    padz = jnp.zeros((_PAD - _NIDX,), jnp.int32)
    idxr = jnp.concatenate([qt_flat,
                            jnp.zeros((_PADR - _NIDX,), jnp.int32)])
    idxe = jnp.concatenate([xt_flat, padz, qt_flat + 2 * _C, padz])

    gtabT = jnp.pad(graph.T, ((0, 0), (0, _CW - _C)))
    etab = jnp.pad(jnp.concatenate([emb_x, emb_c[:_C, :]], axis=0),
                   ((0, 0), (0, _EW - _E)))

    rv_out, em_out = _sc_gather(gtabT, etab, idxr, idxe)
    reT = em_out[:_NIDX, :_E].T
    ecqT = em_out[_PAD:_PAD + _NIDX, :_E].T

    WA = jnp.concatenate([fn0_W1[:, :2 * _H], fn1_W1[:, :2 * _H]], axis=0)
    WU = jnp.concatenate([fn0_W1[:, 2 * _H:3 * _H],
                          fn1_W1[:, 2 * _H:3 * _H]], axis=0)
    WV = jnp.concatenate([fn0_W1[:, 3 * _H:], fn1_W1[:, 3 * _H:]], axis=0)
    b01 = jnp.concatenate([fn0_b1, fn1_b1]).reshape(2 * _H, 1)
    zH = jnp.zeros((_H, _H), f32)
    W2T = jnp.concatenate([
        jnp.concatenate([fn0_W2, zH], axis=1),
        jnp.concatenate([zH, fn1_W2], axis=1)], axis=0)
    b2c = jnp.concatenate([fn0_b2, fn1_b2]).reshape(2 * _H, 1)
    g01 = jnp.concatenate([fn0_g, fn1_g]).reshape(2 * _H, 1)
    bt01 = jnp.concatenate([fn0_bt, fn1_bt]).reshape(2 * _H, 1)
    WEAo = jnp.concatenate([ea_We, ea_Wa], axis=0)
    bEA = jnp.concatenate([ea_be, ea_ba]).reshape(2 * _H, 1)
    zH2 = jnp.zeros((_H, _H), f32)
    WG = jnp.concatenate([
        jnp.concatenate([gru_Wih[:_H], gru_Whh[:_H]], axis=1),
        jnp.concatenate([gru_Wih[_H:2 * _H], gru_Whh[_H:2 * _H]], axis=1),
        jnp.concatenate([gru_Wih[2 * _H:], zH2], axis=1),
        jnp.concatenate([zH2, gru_Whh[2 * _H:]], axis=1),
    ], axis=0)
    bg = jnp.concatenate([
        gru_bih[:2 * _H] + gru_bhh[:2 * _H],
        gru_bih[2 * _H:], gru_bhh[2 * _H:]]).reshape(4 * _H, 1)

    embcT = jnp.pad(emb_c[:_C, :].T, ((0, 0), (0, _CW - _C)))
    eaw_r = jnp.pad(ea_w, (0, _CW - _C)).reshape(1, _CW)

    args = (qi, reT, ecqT, rv_out, graph, embcT, eaw_r,
            WA, b01, WU, WV, W2T, b2c, g01, bt01,
            WEAo, bEA, WG, bg,
            fs_W1, fs_b1.reshape(_H, 1), fs_W2, fs_b2.reshape(_H, 1),
            fs_g.reshape(_H, 1), fs_bt.reshape(_H, 1),
            Wp.reshape(_H, 1), bp.reshape(1, 1))
    return _tc_call(args)
